# baseline (speedup 1.0000x reference)
import functools

import jax
import jax.numpy as jnp
from jax import lax
from jax.experimental import pallas as pl
from jax.experimental.pallas import tpu as pltpu
from jax.experimental.pallas import tpu_sc as plsc

N = 10000
E = 160000
D = 256
HALF = 128
NSUB = 16
CH = 128
EPAD = 163840
NCH = EPAD // (NSUB * CH)
NCHD = EPAD // (2 * NSUB * CH)
NPAD = 10240
RPT = NPAD // NSUB
NWB = RPT // CH


def _sc_mesh():
    return plsc.VectorSubcoreMesh(core_axis_name="c", subcore_axis_name="s")


def _sc_deg(dst3):
    @functools.partial(
        pl.kernel,
        out_type=jax.ShapeDtypeStruct((2 * NPAD, HALF), jnp.float32),
        mesh=_sc_mesh(),
        scratch_types=[
            pltpu.VMEM((NCHD, CH), jnp.int32),
            pltpu.VMEM((CH, HALF), jnp.float32),
            pltpu.VMEM((CH, HALF), jnp.float32),
            pltpu.VMEM_SHARED((NPAD, HALF), jnp.float32),
        ],
    )
    def k(dst3_hbm, out_hbm, didx_v, ones_v, wb_v, acc_sh):
        c = lax.axis_index("c")
        s = lax.axis_index("s")
        w = c * NSUB + s
        oout = c * NPAD
        pltpu.sync_copy(dst3_hbm.at[w], didx_v)

        def zero(i, _):
            for jj in range(HALF // 16):
                ones_v[i, pl.ds(16 * jj, 16)] = jnp.zeros((16,), jnp.float32)
            return 0

        lax.fori_loop(0, CH, zero, 0)
        for t in range(NWB):
            pltpu.sync_copy(ones_v, acc_sh.at[pl.ds(s * RPT + t * CH, CH)])

        def fill(i, _):
            for jj in range(HALF // 16):
                ones_v[i, pl.ds(16 * jj, 16)] = jnp.ones((16,), jnp.float32)
            return 0

        lax.fori_loop(0, CH, fill, 0)
        plsc.subcore_barrier()

        def chunk(j, _):
            pltpu.sync_copy(ones_v, acc_sh.at[didx_v.at[j]], add=True)
            return 0

        lax.fori_loop(0, NCHD, chunk, 0)
        plsc.subcore_barrier()

        for t in range(NWB):
            r0 = s * RPT + t * CH
            pltpu.sync_copy(acc_sh.at[pl.ds(r0, CH)], wb_v)
            pltpu.sync_copy(wb_v, out_hbm.at[pl.ds(oout + r0, CH)])

    return k(dst3)


def _sc_agg(h_flat, src3, dst_flat):
    @functools.partial(
        pl.kernel,
        out_type=jax.ShapeDtypeStruct((2 * NPAD, HALF), jnp.float32),
        mesh=_sc_mesh(),
        scratch_types=[
            pltpu.VMEM((NCH, CH), jnp.int32),
            pltpu.VMEM((CH,), jnp.int32),
            pltpu.VMEM((CH,), jnp.int32),
            pltpu.VMEM((CH, HALF), jnp.float32),
            pltpu.VMEM((CH, HALF), jnp.float32),
            pltpu.VMEM_SHARED((NPAD, HALF), jnp.float32),
            pltpu.SemaphoreType.DMA,
            pltpu.SemaphoreType.DMA,
            pltpu.SemaphoreType.DMA,
            pltpu.SemaphoreType.DMA,
        ],
    )
    def k(h_hbm, src3_hbm, dst_hbm, out_hbm, gidx_v, didx0, didx1,
          rows0, rows1, acc_sh, semg0, semg1, semd0, semd1):
        c = lax.axis_index("c")
        s = lax.axis_index("s")
        off = c * N
        oout = c * NPAD
        tb = s * (NCH * CH)
        rows = (rows0, rows1)
        didx = (didx0, didx1)
        semg = (semg0, semg1)
        semd = (semd0, semd1)

        pltpu.sync_copy(src3_hbm.at[s], gidx_v)

        def zero(i, _):
            for jj in range(HALF // 16):
                rows0[i, pl.ds(16 * jj, 16)] = jnp.zeros((16,), jnp.float32)
            return 0

        lax.fori_loop(0, CH, zero, 0)

        def xform(i, _):
            for jj in range(CH // 16):
                sl = pl.ds(16 * jj, 16)
                gidx_v[i, sl] = gidx_v[i, sl] + jnp.broadcast_to(off, (16,))
            return 0

        lax.fori_loop(0, NCH, xform, 0)

        for t in range(NWB):
            pltpu.sync_copy(rows0, acc_sh.at[pl.ds(s * RPT + t * CH, CH)])
        plsc.subcore_barrier()

        def gissue(j, b):
            pltpu.async_copy(h_hbm.at[gidx_v.at[j]], rows[b], semg[b])
            pltpu.async_copy(dst_hbm.at[pl.ds(tb + j * CH, CH)], didx[b],
                             semd[b])

        def gwait(b):
            pltpu.make_async_copy(h_hbm.at[pl.ds(0, CH)], rows[b],
                                  semg[b]).wait()
            pltpu.make_async_copy(dst_hbm.at[pl.ds(0, CH)], didx[b],
                                  semd[b]).wait()

        gissue(0, 0)

        def pair(k2, _):
            j = 2 * k2
            gwait(0)
            gissue(j + 1, 1)
            pltpu.sync_copy(rows0, acc_sh.at[didx0], add=True)
            gwait(1)

            @pl.when(k2 < NCH // 2 - 1)
            def _():
                gissue(j + 2, 0)

            pltpu.sync_copy(rows1, acc_sh.at[didx1], add=True)
            return 0

        lax.fori_loop(0, NCH // 2, pair, 0)
        plsc.subcore_barrier()

        for t in range(NWB):
            b = t % 2
            if t >= 2:
                pltpu.make_async_copy(rows[b], out_hbm.at[pl.ds(0, CH)],
                                      semg[b]).wait()
            r0 = s * RPT + t * CH
            pltpu.sync_copy(acc_sh.at[pl.ds(r0, CH)], rows[b])
            pltpu.async_copy(rows[b], out_hbm.at[pl.ds(oout + r0, CH)],
                             semg[b])
        for b in (0, 1):
            pltpu.make_async_copy(rows[b], out_hbm.at[pl.ds(0, CH)],
                                  semg[b]).wait()

    return k(h_flat, src3, dst_flat)


BM = 1000


def _tc_layer_body(a_ref, h_ref, deg_ref, wl_ref, bl_ref, wr_ref, o_ref):
    r = 1.0 / jnp.maximum(deg_ref[0][:, 0:1] + deg_ref[1][:, 0:1], 1.0)
    acc = jnp.dot(a_ref[0] * r, wl_ref[0:HALF, :],
                  preferred_element_type=jnp.float32)
    acc += jnp.dot(a_ref[1] * r, wl_ref[HALF:D, :],
                   preferred_element_type=jnp.float32)
    acc += jnp.dot(h_ref[0], wr_ref[0:HALF, :],
                   preferred_element_type=jnp.float32)
    acc += jnp.dot(h_ref[1], wr_ref[HALF:D, :],
                   preferred_element_type=jnp.float32)
    acc += bl_ref[0, :][None, :]
    acc = jnp.maximum(acc, 0.0)
    o_ref[0] = acc[:, 0:HALF]
    o_ref[1] = acc[:, HALF:D]


def _tc_layer(aggs, h, deg, Wl, bl, Wr):
    return pl.pallas_call(
        _tc_layer_body,
        grid=(N // BM,),
        in_specs=[
            pl.BlockSpec((2, BM, HALF), lambda i: (0, i, 0)),
            pl.BlockSpec((2, BM, HALF), lambda i: (0, i, 0)),
            pl.BlockSpec((2, BM, HALF), lambda i: (0, i, 0)),
            pl.BlockSpec((D, D), lambda i: (0, 0)),
            pl.BlockSpec((1, D), lambda i: (0, 0)),
            pl.BlockSpec((D, D), lambda i: (0, 0)),
        ],
        out_specs=pl.BlockSpec((2, BM, HALF), lambda i: (0, i, 0)),
        out_shape=jax.ShapeDtypeStruct((2, N, HALF), jnp.float32),
    )(aggs, h, deg, Wl, bl, Wr)


def _tc_final_body(h_ref, w_ref, b_ref, o_ref):
    acc = jnp.dot(h_ref[0], w_ref[0:HALF, :],
                  preferred_element_type=jnp.float32)
    acc += jnp.dot(h_ref[1], w_ref[HALF:D, :],
                   preferred_element_type=jnp.float32)
    o_ref[...] = acc + b_ref[0, :][None, :]


def _tc_final(h, Wlin, blin):
    return pl.pallas_call(
        _tc_final_body,
        grid=(N // BM,),
        in_specs=[
            pl.BlockSpec((2, BM, HALF), lambda i: (0, i, 0)),
            pl.BlockSpec((D, D), lambda i: (0, 0)),
            pl.BlockSpec((1, D), lambda i: (0, 0)),
        ],
        out_specs=pl.BlockSpec((BM, D), lambda i: (i, 0)),
        out_shape=jax.ShapeDtypeStruct((N, D), jnp.float32),
    )(h, Wlin, blin)


def kernel(x, edge_index, Wl0, bl0, Wr0, Wl1, bl1, Wr1, Wl2, bl2, Wr2,
           Wlin, blin):
    src = edge_index[0]
    dst = edge_index[1]
    src_p = jnp.concatenate([src, jnp.zeros((EPAD - E,), jnp.int32)])
    dst_p = jnp.concatenate([dst, jnp.full((EPAD - E,), N, jnp.int32)])
    src3 = src_p.reshape(NSUB, NCH, CH)
    dst3d = dst_p.reshape(2 * NSUB, NCHD, CH)

    h = x.reshape(N, 2, HALF).transpose(1, 0, 2)
    deg = _sc_deg(dst3d).reshape(2, NPAD, HALF)
    for (Wl, bl, Wr) in ((Wl0, bl0, Wr0), (Wl1, bl1, Wr1), (Wl2, bl2, Wr2)):
        aggs = _sc_agg(h.reshape(2 * N, HALF), src3, dst_p).reshape(
            2, NPAD, HALF)
        h = _tc_layer(aggs, h, deg, Wl, bl.reshape(1, D), Wr)
    return _tc_final(h, Wlin, blin.reshape(1, D))


# speedup vs baseline: 1.0029x; 1.0029x over previous
import functools

import jax
import jax.numpy as jnp
from jax import lax
from jax.experimental import pallas as pl
from jax.experimental.pallas import tpu as pltpu
from jax.experimental.pallas import tpu_sc as plsc

N = 10000
E = 160000
D = 256
HALF = 128
NSUB = 16
CH = 128
EPAD = 163840
NCH = EPAD // (NSUB * CH)
NCHD = EPAD // (2 * NSUB * CH)
NPAD = 10240
RPT = NPAD // NSUB
NWB = RPT // CH


def _sc_mesh():
    return plsc.VectorSubcoreMesh(core_axis_name="c", subcore_axis_name="s")


def _sc_deg(dst3):
    @functools.partial(
        pl.kernel,
        out_type=jax.ShapeDtypeStruct((2 * NPAD, HALF), jnp.float32),
        mesh=_sc_mesh(),
        scratch_types=[
            pltpu.VMEM((NCHD, CH), jnp.int32),
            pltpu.VMEM((CH, HALF), jnp.float32),
            pltpu.VMEM((CH, HALF), jnp.float32),
            pltpu.VMEM_SHARED((NPAD, HALF), jnp.float32),
        ],
    )
    def k(dst3_hbm, out_hbm, didx_v, ones_v, wb_v, acc_sh):
        c = lax.axis_index("c")
        s = lax.axis_index("s")
        w = c * NSUB + s
        oout = c * NPAD
        pltpu.sync_copy(dst3_hbm.at[w], didx_v)

        def zero(i, _):
            for jj in range(HALF // 16):
                ones_v[i, pl.ds(16 * jj, 16)] = jnp.zeros((16,), jnp.float32)
            return 0

        lax.fori_loop(0, CH, zero, 0)
        for t in range(NWB):
            pltpu.sync_copy(ones_v, acc_sh.at[pl.ds(s * RPT + t * CH, CH)])

        def fill(i, _):
            for jj in range(HALF // 16):
                ones_v[i, pl.ds(16 * jj, 16)] = jnp.ones((16,), jnp.float32)
            return 0

        lax.fori_loop(0, CH, fill, 0)
        plsc.subcore_barrier()

        def chunk(j, _):
            pltpu.sync_copy(ones_v, acc_sh.at[didx_v.at[j]], add=True)
            return 0

        lax.fori_loop(0, NCHD, chunk, 0)
        plsc.subcore_barrier()

        for t in range(NWB):
            r0 = s * RPT + t * CH
            pltpu.sync_copy(acc_sh.at[pl.ds(r0, CH)], wb_v)
            pltpu.sync_copy(wb_v, out_hbm.at[pl.ds(oout + r0, CH)])

    return k(dst3)


def _sc_agg(h_flat, src3, dst_flat):
    @functools.partial(
        pl.kernel,
        out_type=jax.ShapeDtypeStruct((2 * NPAD, HALF), jnp.float32),
        mesh=_sc_mesh(),
        scratch_types=[
            pltpu.VMEM((NCH, CH), jnp.int32),
            pltpu.VMEM((CH,), jnp.int32),
            pltpu.VMEM((CH,), jnp.int32),
            pltpu.VMEM((CH, HALF), jnp.float32),
            pltpu.VMEM((CH, HALF), jnp.float32),
            pltpu.VMEM_SHARED((NPAD, HALF), jnp.float32),
            pltpu.SemaphoreType.DMA,
            pltpu.SemaphoreType.DMA,
            pltpu.SemaphoreType.DMA,
            pltpu.SemaphoreType.DMA,
            pltpu.SemaphoreType.DMA,
            pltpu.SemaphoreType.DMA,
        ],
    )
    def k(h_hbm, src3_hbm, dst_hbm, out_hbm, gidx_v, didx0, didx1,
          rows0, rows1, acc_sh, semg0, semg1, semd0, semd1, sems0, sems1):
        c = lax.axis_index("c")
        s = lax.axis_index("s")
        off = c * N
        oout = c * NPAD
        tb = s * (NCH * CH)
        rows = (rows0, rows1)
        didx = (didx0, didx1)
        semg = (semg0, semg1)
        semd = (semd0, semd1)

        pltpu.sync_copy(src3_hbm.at[s], gidx_v)

        def zero(i, _):
            for jj in range(HALF // 16):
                rows0[i, pl.ds(16 * jj, 16)] = jnp.zeros((16,), jnp.float32)
            return 0

        lax.fori_loop(0, CH, zero, 0)

        def xform(i, _):
            for jj in range(CH // 16):
                sl = pl.ds(16 * jj, 16)
                gidx_v[i, sl] = gidx_v[i, sl] + jnp.broadcast_to(off, (16,))
            return 0

        lax.fori_loop(0, NCH, xform, 0)

        for t in range(NWB):
            pltpu.sync_copy(rows0, acc_sh.at[pl.ds(s * RPT + t * CH, CH)])
        plsc.subcore_barrier()

        def gissue(j, b):
            pltpu.async_copy(h_hbm.at[gidx_v.at[j]], rows[b], semg[b])
            pltpu.async_copy(dst_hbm.at[pl.ds(tb + j * CH, CH)], didx[b],
                             semd[b])

        def gwait(b):
            pltpu.make_async_copy(h_hbm.at[pl.ds(0, CH)], rows[b],
                                  semg[b]).wait()
            pltpu.make_async_copy(dst_hbm.at[pl.ds(0, CH)], didx[b],
                                  semd[b]).wait()

        sems = (sems0, sems1)

        def sissue(b):
            pltpu.async_copy(rows[b], acc_sh.at[didx[b]], sems[b], add=True)

        def swait(b):
            pltpu.make_async_copy(rows[b], acc_sh.at[pl.ds(0, CH)],
                                  sems[b]).wait()

        gissue(0, 0)

        def pair(k2, _):
            j = 2 * k2
            gwait(0)

            @pl.when(k2 > 0)
            def _():
                swait(1)

            gissue(j + 1, 1)
            sissue(0)
            gwait(1)
            swait(0)

            @pl.when(k2 < NCH // 2 - 1)
            def _():
                gissue(j + 2, 0)

            sissue(1)
            return 0

        lax.fori_loop(0, NCH // 2, pair, 0)
        swait(1)
        plsc.subcore_barrier()

        for t in range(NWB):
            b = t % 2
            if t >= 2:
                pltpu.make_async_copy(rows[b], out_hbm.at[pl.ds(0, CH)],
                                      semg[b]).wait()
            r0 = s * RPT + t * CH
            pltpu.sync_copy(acc_sh.at[pl.ds(r0, CH)], rows[b])
            pltpu.async_copy(rows[b], out_hbm.at[pl.ds(oout + r0, CH)],
                             semg[b])
        for b in (0, 1):
            pltpu.make_async_copy(rows[b], out_hbm.at[pl.ds(0, CH)],
                                  semg[b]).wait()

    return k(h_flat, src3, dst_flat)


BM = 1000


def _tc_layer_body(a_ref, h_ref, deg_ref, wl_ref, bl_ref, wr_ref, o_ref):
    r = 1.0 / jnp.maximum(deg_ref[0][:, 0:1] + deg_ref[1][:, 0:1], 1.0)
    acc = jnp.dot(a_ref[0] * r, wl_ref[0:HALF, :],
                  preferred_element_type=jnp.float32)
    acc += jnp.dot(a_ref[1] * r, wl_ref[HALF:D, :],
                   preferred_element_type=jnp.float32)
    acc += jnp.dot(h_ref[0], wr_ref[0:HALF, :],
                   preferred_element_type=jnp.float32)
    acc += jnp.dot(h_ref[1], wr_ref[HALF:D, :],
                   preferred_element_type=jnp.float32)
    acc += bl_ref[0, :][None, :]
    acc = jnp.maximum(acc, 0.0)
    o_ref[0] = acc[:, 0:HALF]
    o_ref[1] = acc[:, HALF:D]


def _tc_layer(aggs, h, deg, Wl, bl, Wr):
    return pl.pallas_call(
        _tc_layer_body,
        grid=(N // BM,),
        in_specs=[
            pl.BlockSpec((2, BM, HALF), lambda i: (0, i, 0)),
            pl.BlockSpec((2, BM, HALF), lambda i: (0, i, 0)),
            pl.BlockSpec((2, BM, HALF), lambda i: (0, i, 0)),
            pl.BlockSpec((D, D), lambda i: (0, 0)),
            pl.BlockSpec((1, D), lambda i: (0, 0)),
            pl.BlockSpec((D, D), lambda i: (0, 0)),
        ],
        out_specs=pl.BlockSpec((2, BM, HALF), lambda i: (0, i, 0)),
        out_shape=jax.ShapeDtypeStruct((2, N, HALF), jnp.float32),
    )(aggs, h, deg, Wl, bl, Wr)


def _tc_final_body(h_ref, w_ref, b_ref, o_ref):
    acc = jnp.dot(h_ref[0], w_ref[0:HALF, :],
                  preferred_element_type=jnp.float32)
    acc += jnp.dot(h_ref[1], w_ref[HALF:D, :],
                   preferred_element_type=jnp.float32)
    o_ref[...] = acc + b_ref[0, :][None, :]


def _tc_final(h, Wlin, blin):
    return pl.pallas_call(
        _tc_final_body,
        grid=(N // BM,),
        in_specs=[
            pl.BlockSpec((2, BM, HALF), lambda i: (0, i, 0)),
            pl.BlockSpec((D, D), lambda i: (0, 0)),
            pl.BlockSpec((1, D), lambda i: (0, 0)),
        ],
        out_specs=pl.BlockSpec((BM, D), lambda i: (i, 0)),
        out_shape=jax.ShapeDtypeStruct((N, D), jnp.float32),
    )(h, Wlin, blin)


def kernel(x, edge_index, Wl0, bl0, Wr0, Wl1, bl1, Wr1, Wl2, bl2, Wr2,
           Wlin, blin):
    src = edge_index[0]
    dst = edge_index[1]
    src_p = jnp.concatenate([src, jnp.zeros((EPAD - E,), jnp.int32)])
    dst_p = jnp.concatenate([dst, jnp.full((EPAD - E,), N, jnp.int32)])
    src3 = src_p.reshape(NSUB, NCH, CH)
    dst3d = dst_p.reshape(2 * NSUB, NCHD, CH)

    h = x.reshape(N, 2, HALF).transpose(1, 0, 2)
    deg = _sc_deg(dst3d).reshape(2, NPAD, HALF)
    for (Wl, bl, Wr) in ((Wl0, bl0, Wr0), (Wl1, bl1, Wr1), (Wl2, bl2, Wr2)):
        aggs = _sc_agg(h.reshape(2 * N, HALF), src3, dst_p).reshape(
            2, NPAD, HALF)
        h = _tc_layer(aggs, h, deg, Wl, bl.reshape(1, D), Wr)
    return _tc_final(h, Wlin, blin.reshape(1, D))


# speedup vs baseline: 1.0207x; 1.0177x over previous
import functools

import jax
import jax.numpy as jnp
from jax import lax
from jax.experimental import pallas as pl
from jax.experimental.pallas import tpu as pltpu
from jax.experimental.pallas import tpu_sc as plsc

N = 10000
E = 160000
D = 256
HALF = 128
NSUB = 16
CH = 80
EPT = E // NSUB
NCHUNK = EPT // CH
NPAD = 10240
RPT = NPAD // NSUB
NWB = RPT // CH


def _sc_mesh():
    return plsc.VectorSubcoreMesh(core_axis_name="c", subcore_axis_name="s")


def _sc_deg(dst):
    @functools.partial(
        pl.kernel,
        out_type=jax.ShapeDtypeStruct((NPAD, HALF), jnp.float32),
        mesh=_sc_mesh(),
        scratch_types=[
            pltpu.VMEM((CH,), jnp.int32),
            pltpu.VMEM((CH, HALF), jnp.float32),
            pltpu.VMEM_SHARED((NPAD, HALF), jnp.float32),
        ],
    )
    def k(dst_hbm, out_hbm, didx_v, ones_v, acc_sh):
        c = lax.axis_index("c")
        s = lax.axis_index("s")
        t0 = s * EPT

        def zero(i, _):
            for jj in range(HALF // 16):
                ones_v[i, pl.ds(16 * jj, 16)] = jnp.zeros((16,), jnp.float32)
            return 0

        lax.fori_loop(0, CH, zero, 0)
        for t in range(NWB):
            pltpu.sync_copy(ones_v, acc_sh.at[pl.ds(s * RPT + t * CH, CH)])

        def fill(i, _):
            for jj in range(HALF // 16):
                ones_v[i, pl.ds(16 * jj, 16)] = jnp.ones((16,), jnp.float32)
            return 0

        lax.fori_loop(0, CH, fill, 0)
        plsc.subcore_barrier()

        @pl.when(c == 0)
        def _():
            def chunk(j, _):
                pltpu.sync_copy(dst_hbm.at[pl.ds(t0 + j * CH, CH)], didx_v)
                pltpu.sync_copy(ones_v, acc_sh.at[didx_v], add=True)
                return 0

            lax.fori_loop(0, NCHUNK, chunk, 0)

        plsc.subcore_barrier()

        @pl.when(c == 0)
        def _():
            for t in range(NWB):
                r0 = s * RPT + t * CH
                pltpu.sync_copy(acc_sh.at[pl.ds(r0, CH)], ones_v)
                pltpu.sync_copy(ones_v, out_hbm.at[pl.ds(r0, CH)])

    return k(dst)


def _sc_agg(h_flat, src, dst):
    @functools.partial(
        pl.kernel,
        out_type=jax.ShapeDtypeStruct((2 * NPAD, HALF), jnp.float32),
        mesh=_sc_mesh(),
        scratch_types=[
            pltpu.VMEM((CH,), jnp.int32),
            pltpu.VMEM((CH,), jnp.int32),
            pltpu.VMEM((CH, HALF), jnp.float32),
            pltpu.VMEM_SHARED((NPAD, HALF), jnp.float32),
            pltpu.SemaphoreType.DMA,
        ],
    )
    def k(h_hbm, src_hbm, dst_hbm, out_hbm, gidx_v, didx_v, rows_v,
          acc_sh, sem):
        c = lax.axis_index("c")
        s = lax.axis_index("s")
        t0 = s * EPT
        off = c * N
        oout = c * NPAD

        def zero(i, _):
            for jj in range(HALF // 16):
                rows_v[i, pl.ds(16 * jj, 16)] = jnp.zeros((16,), jnp.float32)
            return 0

        lax.fori_loop(0, CH, zero, 0)
        for t in range(NWB):
            pltpu.sync_copy(rows_v, acc_sh.at[pl.ds(s * RPT + t * CH, CH)])
        plsc.subcore_barrier()

        def chunk(j, _):
            e0 = t0 + j * CH
            pltpu.sync_copy(src_hbm.at[pl.ds(e0, CH)], gidx_v)
            pltpu.sync_copy(dst_hbm.at[pl.ds(e0, CH)], didx_v)

            def xform(kk, _):
                sl = pl.ds(16 * kk, 16)
                gidx_v[sl] = gidx_v[sl] + jnp.broadcast_to(off, (16,))
                return 0

            lax.fori_loop(0, CH // 16, xform, 0)
            pltpu.async_copy(h_hbm.at[gidx_v], rows_v, sem).wait()
            pltpu.sync_copy(rows_v, acc_sh.at[didx_v], add=True)
            return 0

        lax.fori_loop(0, NCHUNK, chunk, 0)
        plsc.subcore_barrier()

        for t in range(NWB):
            r0 = s * RPT + t * CH
            pltpu.sync_copy(acc_sh.at[pl.ds(r0, CH)], rows_v)
            pltpu.sync_copy(rows_v, out_hbm.at[pl.ds(oout + r0, CH)])

    return k(h_flat, src, dst)


BM = 1000


def _tc_layer_body(a_ref, h_ref, deg_ref, wl_ref, bl_ref, wr_ref, o_ref):
    r = 1.0 / jnp.maximum(deg_ref[:, 0:1], 1.0)
    acc = jnp.dot(a_ref[0] * r, wl_ref[0:HALF, :],
                  preferred_element_type=jnp.float32)
    acc += jnp.dot(a_ref[1] * r, wl_ref[HALF:D, :],
                   preferred_element_type=jnp.float32)
    acc += jnp.dot(h_ref[0], wr_ref[0:HALF, :],
                   preferred_element_type=jnp.float32)
    acc += jnp.dot(h_ref[1], wr_ref[HALF:D, :],
                   preferred_element_type=jnp.float32)
    acc += bl_ref[0, :][None, :]
    acc = jnp.maximum(acc, 0.0)
    o_ref[0] = acc[:, 0:HALF]
    o_ref[1] = acc[:, HALF:D]


def _tc_layer(aggs, h, deg, Wl, bl, Wr):
    return pl.pallas_call(
        _tc_layer_body,
        grid=(N // BM,),
        in_specs=[
            pl.BlockSpec((2, BM, HALF), lambda i: (0, i, 0)),
            pl.BlockSpec((2, BM, HALF), lambda i: (0, i, 0)),
            pl.BlockSpec((BM, HALF), lambda i: (i, 0)),
            pl.BlockSpec((D, D), lambda i: (0, 0)),
            pl.BlockSpec((1, D), lambda i: (0, 0)),
            pl.BlockSpec((D, D), lambda i: (0, 0)),
        ],
        out_specs=pl.BlockSpec((2, BM, HALF), lambda i: (0, i, 0)),
        out_shape=jax.ShapeDtypeStruct((2, N, HALF), jnp.float32),
    )(aggs, h, deg, Wl, bl, Wr)


def _tc_final_body(h_ref, w_ref, b_ref, o_ref):
    acc = jnp.dot(h_ref[0], w_ref[0:HALF, :],
                  preferred_element_type=jnp.float32)
    acc += jnp.dot(h_ref[1], w_ref[HALF:D, :],
                   preferred_element_type=jnp.float32)
    o_ref[...] = acc + b_ref[0, :][None, :]


def _tc_final(h, Wlin, blin):
    return pl.pallas_call(
        _tc_final_body,
        grid=(N // BM,),
        in_specs=[
            pl.BlockSpec((2, BM, HALF), lambda i: (0, i, 0)),
            pl.BlockSpec((D, D), lambda i: (0, 0)),
            pl.BlockSpec((1, D), lambda i: (0, 0)),
        ],
        out_specs=pl.BlockSpec((BM, D), lambda i: (i, 0)),
        out_shape=jax.ShapeDtypeStruct((N, D), jnp.float32),
    )(h, Wlin, blin)


def kernel(x, edge_index, Wl0, bl0, Wr0, Wl1, bl1, Wr1, Wl2, bl2, Wr2,
           Wlin, blin):
    src = edge_index[0]
    dst = edge_index[1]
    h = x.reshape(N, 2, HALF).transpose(1, 0, 2)
    deg = _sc_deg(dst)
    for (Wl, bl, Wr) in ((Wl0, bl0, Wr0), (Wl1, bl1, Wr1), (Wl2, bl2, Wr2)):
        aggs = _sc_agg(h.reshape(2 * N, HALF), src, dst).reshape(2, NPAD, HALF)
        h = _tc_layer(aggs, h, deg, Wl, bl.reshape(1, D), Wr)
    return _tc_final(h, Wlin, blin.reshape(1, D))
